# baseline (speedup 1.0000x reference)
import functools

import jax
import jax.numpy as jnp
import numpy as np
from jax.experimental import pallas as pl
from jax.experimental.pallas import tpu as pltpu
from jax.experimental.pallas import tpu_sc as plsc

V = 40000
F = 80000
B = 1
S = 5000
SP = 5120
BI = 1024


def _i32(v):
    return np.array([v], np.uint32).view(np.int32)[0]


_K1 = np.uint32(1832780943)
_K2 = np.uint32(270669613)
_K3 = _K1 ^ _K2 ^ np.uint32(0x1BD11BDA)
_KS = (_K1, _K2, _K3)
_ROT1 = (13, 15, 26, 6)
_ROT2 = (17, 29, 16, 24)
_TINY = np.float32(np.finfo(np.float32).tiny)
_CHUNK = 128
_NCHUNK = F // _CHUNK
_BS = 8
_UNROLL = 25


def _tf_rounds(x0, x1, rots):
    for r in rots:
        x0 = x0 + x1
        x1 = jax.lax.shift_left(x1, np.int32(r)) | jax.lax.shift_right_logical(
            x1, np.int32(32 - r))
        x1 = x0 ^ x1
    return x0, x1


def _tf_bits_from_x1(x1):
    inject = ((_KS[1], _KS[2] + np.uint32(1)),
              (_KS[2], _KS[0] + np.uint32(2)),
              (_KS[0], _KS[1] + np.uint32(3)),
              (_KS[1], _KS[2] + np.uint32(4)),
              (_KS[2], _KS[0] + np.uint32(5)))
    rots = (_ROT1, _ROT2, _ROT1, _ROT2, _ROT1)
    r = _ROT1[0]
    x0 = x1 + _i32(_KS[0])
    x1r = jax.lax.shift_left(x1, np.int32(r)) | jax.lax.shift_right_logical(
        x1, np.int32(32 - r))
    x1 = x0 ^ x1r
    x0, x1 = _tf_rounds(x0, x1, _ROT1[1:])
    x0 = x0 + _i32(inject[0][0])
    x1 = x1 + _i32(inject[0][1])
    for (a, b), rr in zip(inject[1:], rots[1:]):
        x0, x1 = _tf_rounds(x0, x1, rr)
        x0 = x0 + _i32(a)
        x1 = x1 + _i32(b)
    return x0 ^ x1


def _score_chunk(ninvp_ref, base_x1, cc):
    x1 = base_x1 + cc * _CHUNK
    bits = _tf_bits_from_x1(x1)
    fb = jax.lax.shift_right_logical(bits, np.int32(9)) | _i32(0x3F800000)
    u = jax.lax.bitcast_convert_type(fb, jnp.float32) - np.float32(1.0)
    ninvp = ninvp_ref[cc]
    return jnp.log2(u) * ninvp


def _sample_body(ninvp_ref, out_ref):
    pid = pl.program_id(0)
    base_x1 = ((pid * _BS
                + jax.lax.broadcasted_iota(jnp.int32, (_BS, _CHUNK), 0)) * F
               + jax.lax.broadcasted_iota(jnp.int32, (_BS, _CHUNK), 1)
               + _i32(_KS[1]))

    def step(c, carry):
        minval, minc = carry
        for k in range(_UNROLL):
            cc = _UNROLL * c + k
            score = _score_chunk(ninvp_ref, base_x1, cc)
            upd = score < minval
            minval = jnp.where(upd, score, minval)
            minc = jnp.where(upd, cc, minc)
        return minval, minc

    init = (jnp.full((_BS, _CHUNK), np.float32(np.inf), jnp.float32),
            jnp.zeros((_BS, _CHUNK), jnp.int32))
    nloop = _NCHUNK // _UNROLL
    minval, minc = jax.lax.fori_loop(0, nloop, step, init)
    for cc in range(nloop * _UNROLL, _NCHUNK):
        score = _score_chunk(ninvp_ref, base_x1, cc)
        upd = score < minval
        minval = jnp.where(upd, score, minval)
        minc = jnp.where(upd, cc, minc)

    minidx = minc * _CHUNK + jax.lax.broadcasted_iota(
        jnp.int32, (_BS, _CHUNK), 1)
    rowmin = jnp.min(minval, axis=1, keepdims=True)
    cand = jnp.where(minval == rowmin, minidx, np.int32(2**31 - 1))
    out_ref[...] = jnp.min(cand, axis=1, keepdims=True)


def _sample_face_idx(ninvp):
    out = pl.pallas_call(
        _sample_body,
        grid=(S // _BS,),
        in_specs=[pl.BlockSpec((_NCHUNK, _BS, _CHUNK), lambda i: (0, 0, 0))],
        out_specs=pl.BlockSpec((_BS, 1), lambda i: (i, 0)),
        out_shape=jax.ShapeDtypeStruct((S, 1), jnp.int32),
        compiler_params=pltpu.CompilerParams(
            dimension_semantics=("arbitrary",)),
    )(ninvp)
    return out.reshape(S)


def _chamfer_body(s_ref, tT_ref, rm_ref, cm_ref):
    i = pl.program_id(0)
    s = s_ref[...]
    tT = tT_ref[...]
    d = jnp.zeros((BI, SP), jnp.float32)
    for c in range(3):
        diff = s[:, c:c + 1] - tT[c:c + 1, :]
        d = d + diff * diff
    rm = jnp.min(d, axis=1)
    rm_ref[...] = rm.reshape(BI // 128, 128)
    cm = jnp.min(d, axis=0)
    cm = cm.reshape(SP // 128, 128)

    @pl.when(i == 0)
    def _():
        cm_ref[...] = cm

    @pl.when(i != 0)
    def _():
        cm_ref[...] = jnp.minimum(cm_ref[...], cm)


def _chamfer(sample_pts, trg_pts):
    big_s = 1e9
    big_t = -1e9
    s = jnp.full((SP, 8), big_s, jnp.float32).at[:S, :3].set(sample_pts)
    t = jnp.full((SP, 8), big_t, jnp.float32).at[:S, :3].set(trg_pts)
    tT = t.T.reshape(8, SP)

    rm, cm = pl.pallas_call(
        _chamfer_body,
        grid=(SP // BI,),
        in_specs=[
            pl.BlockSpec((BI, 8), lambda i: (i, 0)),
            pl.BlockSpec((8, SP), lambda i: (0, 0)),
        ],
        out_specs=[
            pl.BlockSpec((BI // 128, 128), lambda i: (i, 0)),
            pl.BlockSpec((SP // 128, 128), lambda i: (0, 0)),
        ],
        out_shape=[
            jax.ShapeDtypeStruct((SP // 128, 128), jnp.float32),
            jax.ShapeDtypeStruct((SP // 128, 128), jnp.float32),
        ],
        compiler_params=pltpu.CompilerParams(
            dimension_semantics=("arbitrary",)),
    )(s, tT)
    return rm.reshape(SP), cm.reshape(SP)


def _prep_body(vc_ref, el_ref, ninvp_ref, pay_ref, esum_ref):
    vs = [vc_ref[k] for k in range(9)]
    v0 = vs[0:3]
    v1 = vs[3:6]
    v2 = vs[6:9]
    e1 = [v1[k] - v0[k] for k in range(3)]
    e2 = [v2[k] - v0[k] for k in range(3)]
    cx = e1[1] * e2[2] - e1[2] * e2[1]
    cy = e1[2] * e2[0] - e1[0] * e2[2]
    cz = e1[0] * e2[1] - e1[1] * e2[0]
    c2sum = cx * cx + cy * cy + cz * cz
    areas = 0.5 * jnp.sqrt(c2sum)
    asum = jnp.sum(areas)
    p = areas / asum + np.float32(1e-12)
    nv = np.float32(-1.0) / p
    ninvp_ref[...] = jnp.broadcast_to(nv[:, None, :],
                                      (_NCHUNK, _BS, _CHUNK))

    def cot(a, b, c):
        d1 = [b[k] - a[k] for k in range(3)]
        d2 = [c[k] - a[k] for k in range(3)]
        cosang = d1[0] * d2[0] + d1[1] * d2[1] + d1[2] * d2[2]
        sx = d1[1] * d2[2] - d1[2] * d2[1]
        sy = d1[2] * d2[0] - d1[0] * d2[2]
        sz = d1[0] * d2[1] - d1[1] * d2[0]
        sinang = jnp.sqrt(sx * sx + sy * sy + sz * sz)
        return cosang / (sinang + np.float32(1e-12))

    c0 = cot(v0, v1, v2)
    c1 = cot(v1, v2, v0)
    c2 = cot(v2, v0, v1)
    for k in range(3):
        pay_ref[k, 0] = c1 * v2[k] + c2 * v1[k]
        pay_ref[k, 1] = c0 * v2[k] + c2 * v0[k]
        pay_ref[k, 2] = c0 * v1[k] + c1 * v0[k]
    wsplat = jnp.zeros_like(c0)
    pay_ref[3, 0] = c1 + c2 + wsplat
    pay_ref[3, 1] = c0 + c2 + wsplat
    pay_ref[3, 2] = c0 + c1 + wsplat
    L = el_ref[...]

    def sq(d):
        n = jnp.sqrt(d[0] * d[0] + d[1] * d[1] + d[2] * d[2]) - L
        return n * n

    d01 = [v0[k] - v1[k] for k in range(3)]
    d12 = [v1[k] - v2[k] for k in range(3)]
    d02 = [v0[k] - v2[k] for k in range(3)]
    esum_ref[...] = jnp.sum(sq(d01) + sq(d12) + sq(d02)).reshape(1, 1)


def _prep(vcomp, edge_len):
    return pl.pallas_call(
        _prep_body,
        in_specs=[
            pl.BlockSpec(memory_space=pltpu.VMEM),
            pl.BlockSpec(memory_space=pltpu.VMEM),
        ],
        out_specs=[
            pl.BlockSpec(memory_space=pltpu.VMEM),
            pl.BlockSpec(memory_space=pltpu.VMEM),
            pl.BlockSpec(memory_space=pltpu.VMEM),
        ],
        out_shape=[
            jax.ShapeDtypeStruct((_NCHUNK, _BS, _CHUNK), jnp.float32),
            jax.ShapeDtypeStruct((4, 3, _NCHUNK, _CHUNK), jnp.float32),
            jax.ShapeDtypeStruct((1, 1), jnp.float32),
        ],
    )(vcomp, edge_len.reshape(1, 1))


_VP = 40960
_NW = 32
_P3F = 245760
_EPW = 4 * _P3F // _NW
_NCK = _EPW // 128
_ACC = 4 * _VP


def _lap_scatter_body(vals_hbm, idx_hbm, zeros_hbm, out_hbm,
                      vals_v, idx_v, acc_sh):
    c = jax.lax.axis_index("c")
    s = jax.lax.axis_index("s")
    wid = s * 2 + c
    aslc = _ACC // 16
    pltpu.sync_copy(zeros_hbm.at[pl.ds(s * aslc, aslc)],
                    acc_sh.at[pl.ds(s * aslc, aslc)])
    plsc.subcore_barrier()
    pltpu.sync_copy(vals_hbm.at[pl.ds(wid * _EPW, _EPW)], vals_v)
    pltpu.sync_copy(idx_hbm.at[pl.ds(wid * _NCK, _NCK)], idx_v)

    def chunk(j, carry):
        pltpu.sync_copy(vals_v.at[pl.ds(j * 128, 128)],
                        acc_sh.at[idx_v.at[j]], add=True)
        return carry

    jax.lax.fori_loop(0, _NCK, chunk, 0)
    plsc.subcore_barrier()
    pltpu.sync_copy(acc_sh.at[pl.ds(s * aslc, aslc)],
                    out_hbm.at[c].at[pl.ds(s * aslc, aslc)])


def _lap_scatter(vals_flat, idx2, zeros):
    kfn = pl.kernel(
        _lap_scatter_body,
        out_type=jax.ShapeDtypeStruct((2, _ACC), jnp.float32),
        mesh=plsc.VectorSubcoreMesh(core_axis_name="c", subcore_axis_name="s",
                                    num_cores=2, num_subcores=16),
        scratch_types=[
            pltpu.VMEM((_EPW,), jnp.float32),
            pltpu.VMEM((_NCK, 128), jnp.int32),
            pltpu.VMEM_SHARED((_ACC,), jnp.float32),
        ],
    )
    return kfn(vals_flat, idx2, zeros)


def _cot(a, b, c):
    e1 = b - a
    e2 = c - a
    cosang = jnp.sum(e1 * e2, axis=-1)
    sinang = jnp.linalg.norm(jnp.cross(e1, e2), axis=-1)
    return cosang / (sinang + 1e-12)


def kernel(verts_src, trg, edge_len, faces):
    f0, f1, f2 = faces[:, 0], faces[:, 1], faces[:, 2]
    v0 = verts_src[f0]
    v1 = verts_src[f1]
    v2 = verts_src[f2]
    vcomp = jnp.concatenate([v0, v1, v2], axis=1).T.reshape(9, _NCHUNK, _CHUNK)
    ninvp, pay, esum = _prep(vcomp, edge_len)
    skey = jax.random.key(42)
    _, ks2, ks3 = jax.random.split(skey, 3)
    face_idx = _sample_face_idx(ninvp)[None, :]
    u = jax.random.uniform(ks2, (B, S, 1), dtype=jnp.float32)
    vv = jax.random.uniform(ks3, (B, S, 1), dtype=jnp.float32)
    su = jnp.sqrt(u)
    w0 = 1.0 - su
    w1 = su * (1.0 - vv)
    w2 = su * vv
    sample_scr = w0 * v0[face_idx] + w1 * v1[face_idx] + w2 * v2[face_idx]
    rm, cm = _chamfer(sample_scr[0], trg[0])
    loss_p0 = jnp.mean(rm[:S]) + jnp.mean(cm[:S])
    loss_n1 = jnp.asarray(1e-5, dtype=jnp.float32)
    idx3 = jnp.concatenate(
        [f0, f1, f2,
         jnp.full((_P3F - 3 * F,), _VP - 1, jnp.int32)]).astype(jnp.int32)
    idx_el = (jnp.arange(4, dtype=jnp.int32)[:, None] * _VP
              + idx3[None, :]).reshape(-1)
    vals4 = jnp.concatenate(
        [pay.reshape(4, 3 * F),
         jnp.zeros((4, _P3F - 3 * F), jnp.float32)], axis=1)
    acc2 = _lap_scatter(vals4.reshape(-1),
                        idx_el.reshape(_NW * _NCK, 128),
                        jnp.zeros((_ACC,), jnp.float32))
    A = (acc2[0] + acc2[1]).reshape(4, _VP)
    Lv = A[:3, :V].T
    wsum = A[3, :V]
    safe = jnp.where(wsum > 0, wsum, 1.0)
    norm_w = jnp.where(wsum > 0, 1.0 / safe, 0.0)
    lap = Lv * norm_w[:, None] - verts_src
    loss_laplacian = jnp.mean(jnp.linalg.norm(lap, axis=1))
    loss_edge = esum[0, 0] / np.float32(3 * F)
    return jnp.stack([loss_p0, loss_n1, loss_laplacian, loss_edge])


# speedup vs baseline: 1.0304x; 1.0304x over previous
import functools

import jax
import jax.numpy as jnp
import numpy as np
from jax.experimental import pallas as pl
from jax.experimental.pallas import tpu as pltpu
from jax.experimental.pallas import tpu_sc as plsc

V = 40000
F = 80000
B = 1
S = 5000
SP = 5120
BI = 1024


def _i32(v):
    return np.array([v], np.uint32).view(np.int32)[0]


_K1 = np.uint32(1832780943)
_K2 = np.uint32(270669613)
_K3 = _K1 ^ _K2 ^ np.uint32(0x1BD11BDA)
_KS = (_K1, _K2, _K3)
_ROT1 = (13, 15, 26, 6)
_ROT2 = (17, 29, 16, 24)
_TINY = np.float32(np.finfo(np.float32).tiny)
_CHUNK = 640
_NCHUNK = F // _CHUNK
_BS = 8
_UNROLL = 6


def _tf_rounds(x0, x1, rots):
    for r in rots:
        x0 = x0 + x1
        x1 = jax.lax.shift_left(x1, np.int32(r)) | jax.lax.shift_right_logical(
            x1, np.int32(32 - r))
        x1 = x0 ^ x1
    return x0, x1


def _tf_bits_from_x1(x1):
    inject = ((_KS[1], _KS[2] + np.uint32(1)),
              (_KS[2], _KS[0] + np.uint32(2)),
              (_KS[0], _KS[1] + np.uint32(3)),
              (_KS[1], _KS[2] + np.uint32(4)),
              (_KS[2], _KS[0] + np.uint32(5)))
    rots = (_ROT1, _ROT2, _ROT1, _ROT2, _ROT1)
    r = _ROT1[0]
    x0 = x1 + _i32(_KS[0])
    x1r = jax.lax.shift_left(x1, np.int32(r)) | jax.lax.shift_right_logical(
        x1, np.int32(32 - r))
    x1 = x0 ^ x1r
    x0, x1 = _tf_rounds(x0, x1, _ROT1[1:])
    x0 = x0 + _i32(inject[0][0])
    x1 = x1 + _i32(inject[0][1])
    for (a, b), rr in zip(inject[1:], rots[1:]):
        x0, x1 = _tf_rounds(x0, x1, rr)
        x0 = x0 + _i32(a)
        x1 = x1 + _i32(b)
    return x0 ^ x1


def _score_chunk(ninvp_ref, base_x1, cc):
    x1 = base_x1 + cc * _CHUNK
    bits = _tf_bits_from_x1(x1)
    fb = jax.lax.shift_right_logical(bits, np.int32(9)) | _i32(0x3F800000)
    u = jax.lax.bitcast_convert_type(fb, jnp.float32) - np.float32(1.0)
    ninvp = ninvp_ref[cc]
    return jnp.log2(u) * ninvp


def _sample_body(ninvp_ref, out_ref):
    pid = pl.program_id(0)
    base_x1 = ((pid * _BS
                + jax.lax.broadcasted_iota(jnp.int32, (_BS, _CHUNK), 0)) * F
               + jax.lax.broadcasted_iota(jnp.int32, (_BS, _CHUNK), 1)
               + _i32(_KS[1]))

    def step(c, carry):
        minval, minc = carry
        for k in range(_UNROLL):
            cc = _UNROLL * c + k
            score = _score_chunk(ninvp_ref, base_x1, cc)
            upd = score < minval
            minval = jnp.where(upd, score, minval)
            minc = jnp.where(upd, cc, minc)
        return minval, minc

    init = (jnp.full((_BS, _CHUNK), np.float32(np.inf), jnp.float32),
            jnp.zeros((_BS, _CHUNK), jnp.int32))
    nloop = _NCHUNK // _UNROLL
    minval, minc = jax.lax.fori_loop(0, nloop, step, init)
    for cc in range(nloop * _UNROLL, _NCHUNK):
        score = _score_chunk(ninvp_ref, base_x1, cc)
        upd = score < minval
        minval = jnp.where(upd, score, minval)
        minc = jnp.where(upd, cc, minc)

    minidx = minc * _CHUNK + jax.lax.broadcasted_iota(
        jnp.int32, (_BS, _CHUNK), 1)
    rowmin = jnp.min(minval, axis=1, keepdims=True)
    cand = jnp.where(minval == rowmin, minidx, np.int32(2**31 - 1))
    out_ref[...] = jnp.min(cand, axis=1, keepdims=True)


def _sample_face_idx(ninvp):
    out = pl.pallas_call(
        _sample_body,
        grid=(S // _BS,),
        in_specs=[pl.BlockSpec((_NCHUNK, _BS, _CHUNK), lambda i: (0, 0, 0))],
        out_specs=pl.BlockSpec((_BS, 1), lambda i: (i, 0)),
        out_shape=jax.ShapeDtypeStruct((S, 1), jnp.int32),
        compiler_params=pltpu.CompilerParams(
            dimension_semantics=("arbitrary",)),
    )(ninvp)
    return out.reshape(S)


def _chamfer_body(s_ref, tT_ref, rm_ref, cm_ref):
    i = pl.program_id(0)
    s = s_ref[...]
    tT = tT_ref[...]
    d = jnp.zeros((BI, SP), jnp.float32)
    for c in range(3):
        diff = s[:, c:c + 1] - tT[c:c + 1, :]
        d = d + diff * diff
    rm = jnp.min(d, axis=1)
    rm_ref[...] = rm.reshape(BI // 128, 128)
    cm = jnp.min(d, axis=0)
    cm = cm.reshape(SP // 128, 128)

    @pl.when(i == 0)
    def _():
        cm_ref[...] = cm

    @pl.when(i != 0)
    def _():
        cm_ref[...] = jnp.minimum(cm_ref[...], cm)


def _chamfer(sample_pts, trg_pts):
    big_s = 1e9
    big_t = -1e9
    s = jnp.full((SP, 8), big_s, jnp.float32).at[:S, :3].set(sample_pts)
    t = jnp.full((SP, 8), big_t, jnp.float32).at[:S, :3].set(trg_pts)
    tT = t.T.reshape(8, SP)

    rm, cm = pl.pallas_call(
        _chamfer_body,
        grid=(SP // BI,),
        in_specs=[
            pl.BlockSpec((BI, 8), lambda i: (i, 0)),
            pl.BlockSpec((8, SP), lambda i: (0, 0)),
        ],
        out_specs=[
            pl.BlockSpec((BI // 128, 128), lambda i: (i, 0)),
            pl.BlockSpec((SP // 128, 128), lambda i: (0, 0)),
        ],
        out_shape=[
            jax.ShapeDtypeStruct((SP // 128, 128), jnp.float32),
            jax.ShapeDtypeStruct((SP // 128, 128), jnp.float32),
        ],
        compiler_params=pltpu.CompilerParams(
            dimension_semantics=("arbitrary",)),
    )(s, tT)
    return rm.reshape(SP), cm.reshape(SP)


def _prep_body(vc_ref, el_ref, ninvp_ref, pay_ref, esum_ref):
    vs = [vc_ref[k] for k in range(9)]
    v0 = vs[0:3]
    v1 = vs[3:6]
    v2 = vs[6:9]
    e1 = [v1[k] - v0[k] for k in range(3)]
    e2 = [v2[k] - v0[k] for k in range(3)]
    cx = e1[1] * e2[2] - e1[2] * e2[1]
    cy = e1[2] * e2[0] - e1[0] * e2[2]
    cz = e1[0] * e2[1] - e1[1] * e2[0]
    c2sum = cx * cx + cy * cy + cz * cz
    areas = 0.5 * jnp.sqrt(c2sum)
    asum = jnp.sum(areas)
    p = areas / asum + np.float32(1e-12)
    nv = np.float32(-1.0) / p
    ninvp_ref[...] = jnp.broadcast_to(nv[:, None, :],
                                      (_NCHUNK, _BS, _CHUNK))

    def cot(a, b, c):
        d1 = [b[k] - a[k] for k in range(3)]
        d2 = [c[k] - a[k] for k in range(3)]
        cosang = d1[0] * d2[0] + d1[1] * d2[1] + d1[2] * d2[2]
        sx = d1[1] * d2[2] - d1[2] * d2[1]
        sy = d1[2] * d2[0] - d1[0] * d2[2]
        sz = d1[0] * d2[1] - d1[1] * d2[0]
        sinang = jnp.sqrt(sx * sx + sy * sy + sz * sz)
        return cosang / (sinang + np.float32(1e-12))

    c0 = cot(v0, v1, v2)
    c1 = cot(v1, v2, v0)
    c2 = cot(v2, v0, v1)
    for k in range(3):
        pay_ref[k, 0] = c1 * v2[k] + c2 * v1[k]
        pay_ref[k, 1] = c0 * v2[k] + c2 * v0[k]
        pay_ref[k, 2] = c0 * v1[k] + c1 * v0[k]
    wsplat = jnp.zeros_like(c0)
    pay_ref[3, 0] = c1 + c2 + wsplat
    pay_ref[3, 1] = c0 + c2 + wsplat
    pay_ref[3, 2] = c0 + c1 + wsplat
    L = el_ref[...]

    def sq(d):
        n = jnp.sqrt(d[0] * d[0] + d[1] * d[1] + d[2] * d[2]) - L
        return n * n

    d01 = [v0[k] - v1[k] for k in range(3)]
    d12 = [v1[k] - v2[k] for k in range(3)]
    d02 = [v0[k] - v2[k] for k in range(3)]
    esum_ref[...] = jnp.sum(sq(d01) + sq(d12) + sq(d02)).reshape(1, 1)


def _prep(vcomp, edge_len):
    return pl.pallas_call(
        _prep_body,
        in_specs=[
            pl.BlockSpec(memory_space=pltpu.VMEM),
            pl.BlockSpec(memory_space=pltpu.VMEM),
        ],
        out_specs=[
            pl.BlockSpec(memory_space=pltpu.VMEM),
            pl.BlockSpec(memory_space=pltpu.VMEM),
            pl.BlockSpec(memory_space=pltpu.VMEM),
        ],
        out_shape=[
            jax.ShapeDtypeStruct((_NCHUNK, _BS, _CHUNK), jnp.float32),
            jax.ShapeDtypeStruct((4, 3, _NCHUNK, _CHUNK), jnp.float32),
            jax.ShapeDtypeStruct((1, 1), jnp.float32),
        ],
    )(vcomp, edge_len.reshape(1, 1))


_VP = 40960
_NW = 32
_P3F = 245760
_EPW = 4 * _P3F // _NW
_NCK = _EPW // 128
_ACC = 4 * _VP


def _lap_scatter_body(vals_hbm, idx_hbm, zeros_hbm, out_hbm,
                      vals_v, idx_v, acc_sh):
    c = jax.lax.axis_index("c")
    s = jax.lax.axis_index("s")
    wid = s * 2 + c
    aslc = _ACC // 16
    pltpu.sync_copy(zeros_hbm.at[pl.ds(s * aslc, aslc)],
                    acc_sh.at[pl.ds(s * aslc, aslc)])
    plsc.subcore_barrier()
    pltpu.sync_copy(vals_hbm.at[pl.ds(wid * _EPW, _EPW)], vals_v)
    pltpu.sync_copy(idx_hbm.at[pl.ds(wid * _NCK, _NCK)], idx_v)

    def chunk(j, carry):
        pltpu.sync_copy(vals_v.at[pl.ds(j * 128, 128)],
                        acc_sh.at[idx_v.at[j]], add=True)
        return carry

    jax.lax.fori_loop(0, _NCK, chunk, 0)
    plsc.subcore_barrier()
    pltpu.sync_copy(acc_sh.at[pl.ds(s * aslc, aslc)],
                    out_hbm.at[c].at[pl.ds(s * aslc, aslc)])


def _lap_scatter(vals_flat, idx2, zeros):
    kfn = pl.kernel(
        _lap_scatter_body,
        out_type=jax.ShapeDtypeStruct((2, _ACC), jnp.float32),
        mesh=plsc.VectorSubcoreMesh(core_axis_name="c", subcore_axis_name="s",
                                    num_cores=2, num_subcores=16),
        scratch_types=[
            pltpu.VMEM((_EPW,), jnp.float32),
            pltpu.VMEM((_NCK, 128), jnp.int32),
            pltpu.VMEM_SHARED((_ACC,), jnp.float32),
        ],
    )
    return kfn(vals_flat, idx2, zeros)


def _cot(a, b, c):
    e1 = b - a
    e2 = c - a
    cosang = jnp.sum(e1 * e2, axis=-1)
    sinang = jnp.linalg.norm(jnp.cross(e1, e2), axis=-1)
    return cosang / (sinang + 1e-12)


def kernel(verts_src, trg, edge_len, faces):
    f0, f1, f2 = faces[:, 0], faces[:, 1], faces[:, 2]
    v0 = verts_src[f0]
    v1 = verts_src[f1]
    v2 = verts_src[f2]
    vcomp = jnp.concatenate([v0, v1, v2], axis=1).T.reshape(9, _NCHUNK, _CHUNK)
    ninvp, pay, esum = _prep(vcomp, edge_len)
    skey = jax.random.key(42)
    _, ks2, ks3 = jax.random.split(skey, 3)
    face_idx = _sample_face_idx(ninvp)[None, :]
    u = jax.random.uniform(ks2, (B, S, 1), dtype=jnp.float32)
    vv = jax.random.uniform(ks3, (B, S, 1), dtype=jnp.float32)
    su = jnp.sqrt(u)
    w0 = 1.0 - su
    w1 = su * (1.0 - vv)
    w2 = su * vv
    sample_scr = w0 * v0[face_idx] + w1 * v1[face_idx] + w2 * v2[face_idx]
    rm, cm = _chamfer(sample_scr[0], trg[0])
    loss_p0 = jnp.mean(rm[:S]) + jnp.mean(cm[:S])
    loss_n1 = jnp.asarray(1e-5, dtype=jnp.float32)
    idx3 = jnp.concatenate(
        [f0, f1, f2,
         jnp.full((_P3F - 3 * F,), _VP - 1, jnp.int32)]).astype(jnp.int32)
    idx_el = (jnp.arange(4, dtype=jnp.int32)[:, None] * _VP
              + idx3[None, :]).reshape(-1)
    vals4 = jnp.concatenate(
        [pay.reshape(4, 3 * F),
         jnp.zeros((4, _P3F - 3 * F), jnp.float32)], axis=1)
    acc2 = _lap_scatter(vals4.reshape(-1),
                        idx_el.reshape(_NW * _NCK, 128),
                        jnp.zeros((_ACC,), jnp.float32))
    A = (acc2[0] + acc2[1]).reshape(4, _VP)
    Lv = A[:3, :V].T
    wsum = A[3, :V]
    safe = jnp.where(wsum > 0, wsum, 1.0)
    norm_w = jnp.where(wsum > 0, 1.0 / safe, 0.0)
    lap = Lv * norm_w[:, None] - verts_src
    loss_laplacian = jnp.mean(jnp.linalg.norm(lap, axis=1))
    loss_edge = esum[0, 0] / np.float32(3 * F)
    return jnp.stack([loss_p0, loss_n1, loss_laplacian, loss_edge])


# speedup vs baseline: 1.0356x; 1.0050x over previous
import functools

import jax
import jax.numpy as jnp
import numpy as np
from jax.experimental import pallas as pl
from jax.experimental.pallas import tpu as pltpu
from jax.experimental.pallas import tpu_sc as plsc

V = 40000
F = 80000
B = 1
S = 5000
SP = 5120
BI = 1024


def _i32(v):
    return np.array([v], np.uint32).view(np.int32)[0]


_K1 = np.uint32(1832780943)
_K2 = np.uint32(270669613)
_K3 = _K1 ^ _K2 ^ np.uint32(0x1BD11BDA)
_KS = (_K1, _K2, _K3)
_ROT1 = (13, 15, 26, 6)
_ROT2 = (17, 29, 16, 24)
_TINY = np.float32(np.finfo(np.float32).tiny)
_CHUNK = 640
_NCHUNK = F // _CHUNK
_BS = 8
_UNROLL = 8


def _tf_rounds(x0, x1, rots):
    for r in rots:
        x0 = x0 + x1
        x1 = jax.lax.shift_left(x1, np.int32(r)) | jax.lax.shift_right_logical(
            x1, np.int32(32 - r))
        x1 = x0 ^ x1
    return x0, x1


def _tf_bits_from_x1(x1):
    inject = ((_KS[1], _KS[2] + np.uint32(1)),
              (_KS[2], _KS[0] + np.uint32(2)),
              (_KS[0], _KS[1] + np.uint32(3)),
              (_KS[1], _KS[2] + np.uint32(4)),
              (_KS[2], _KS[0] + np.uint32(5)))
    rots = (_ROT1, _ROT2, _ROT1, _ROT2, _ROT1)
    r = _ROT1[0]
    x0 = x1 + _i32(_KS[0])
    x1r = jax.lax.shift_left(x1, np.int32(r)) | jax.lax.shift_right_logical(
        x1, np.int32(32 - r))
    x1 = x0 ^ x1r
    x0, x1 = _tf_rounds(x0, x1, _ROT1[1:])
    x0 = x0 + _i32(inject[0][0])
    x1 = x1 + _i32(inject[0][1])
    for (a, b), rr in zip(inject[1:], rots[1:]):
        x0, x1 = _tf_rounds(x0, x1, rr)
        x0 = x0 + _i32(a)
        x1 = x1 + _i32(b)
    return x0 ^ x1


def _score_chunk(ninvp_ref, base_x1, cc):
    x1 = base_x1 + cc * _CHUNK
    bits = _tf_bits_from_x1(x1)
    fb = jax.lax.shift_right_logical(bits, np.int32(9)) | _i32(0x3F800000)
    u = jax.lax.bitcast_convert_type(fb, jnp.float32) - np.float32(1.0)
    ninvp = ninvp_ref[cc]
    return jnp.log2(u) * ninvp


def _sample_body(ninvp_ref, out_ref):
    pid = pl.program_id(0)
    base_x1 = ((pid * _BS
                + jax.lax.broadcasted_iota(jnp.int32, (_BS, _CHUNK), 0)) * F
               + jax.lax.broadcasted_iota(jnp.int32, (_BS, _CHUNK), 1)
               + _i32(_KS[1]))

    def step(c, carry):
        minval, minc = carry
        for k in range(_UNROLL):
            cc = _UNROLL * c + k
            score = _score_chunk(ninvp_ref, base_x1, cc)
            upd = score < minval
            minval = jnp.where(upd, score, minval)
            minc = jnp.where(upd, cc, minc)
        return minval, minc

    init = (jnp.full((_BS, _CHUNK), np.float32(np.inf), jnp.float32),
            jnp.zeros((_BS, _CHUNK), jnp.int32))
    nloop = _NCHUNK // _UNROLL
    minval, minc = jax.lax.fori_loop(0, nloop, step, init)
    for cc in range(nloop * _UNROLL, _NCHUNK):
        score = _score_chunk(ninvp_ref, base_x1, cc)
        upd = score < minval
        minval = jnp.where(upd, score, minval)
        minc = jnp.where(upd, cc, minc)

    minidx = minc * _CHUNK + jax.lax.broadcasted_iota(
        jnp.int32, (_BS, _CHUNK), 1)
    rowmin = jnp.min(minval, axis=1, keepdims=True)
    cand = jnp.where(minval == rowmin, minidx, np.int32(2**31 - 1))
    out_ref[...] = jnp.min(cand, axis=1, keepdims=True)


def _sample_face_idx(ninvp):
    out = pl.pallas_call(
        _sample_body,
        grid=(S // _BS,),
        in_specs=[pl.BlockSpec((_NCHUNK, _BS, _CHUNK), lambda i: (0, 0, 0))],
        out_specs=pl.BlockSpec((_BS, 1), lambda i: (i, 0)),
        out_shape=jax.ShapeDtypeStruct((S, 1), jnp.int32),
        compiler_params=pltpu.CompilerParams(
            dimension_semantics=("arbitrary",)),
    )(ninvp)
    return out.reshape(S)


def _chamfer_body(s_ref, tT_ref, rm_ref, cm_ref):
    i = pl.program_id(0)
    s = s_ref[...]
    tT = tT_ref[...]
    d = jnp.zeros((BI, SP), jnp.float32)
    for c in range(3):
        diff = s[:, c:c + 1] - tT[c:c + 1, :]
        d = d + diff * diff
    rm = jnp.min(d, axis=1)
    rm_ref[...] = rm.reshape(BI // 128, 128)
    cm = jnp.min(d, axis=0)
    cm = cm.reshape(SP // 128, 128)

    @pl.when(i == 0)
    def _():
        cm_ref[...] = cm

    @pl.when(i != 0)
    def _():
        cm_ref[...] = jnp.minimum(cm_ref[...], cm)


def _chamfer(sample_pts, trg_pts):
    big_s = 1e9
    big_t = -1e9
    s = jnp.full((SP, 8), big_s, jnp.float32).at[:S, :3].set(sample_pts)
    t = jnp.full((SP, 8), big_t, jnp.float32).at[:S, :3].set(trg_pts)
    tT = t.T.reshape(8, SP)

    rm, cm = pl.pallas_call(
        _chamfer_body,
        grid=(SP // BI,),
        in_specs=[
            pl.BlockSpec((BI, 8), lambda i: (i, 0)),
            pl.BlockSpec((8, SP), lambda i: (0, 0)),
        ],
        out_specs=[
            pl.BlockSpec((BI // 128, 128), lambda i: (i, 0)),
            pl.BlockSpec((SP // 128, 128), lambda i: (0, 0)),
        ],
        out_shape=[
            jax.ShapeDtypeStruct((SP // 128, 128), jnp.float32),
            jax.ShapeDtypeStruct((SP // 128, 128), jnp.float32),
        ],
        compiler_params=pltpu.CompilerParams(
            dimension_semantics=("arbitrary",)),
    )(s, tT)
    return rm.reshape(SP), cm.reshape(SP)


def _prep_body(vc_ref, el_ref, ninvp_ref, pay_ref, esum_ref):
    vs = [vc_ref[k] for k in range(9)]
    v0 = vs[0:3]
    v1 = vs[3:6]
    v2 = vs[6:9]
    e1 = [v1[k] - v0[k] for k in range(3)]
    e2 = [v2[k] - v0[k] for k in range(3)]
    cx = e1[1] * e2[2] - e1[2] * e2[1]
    cy = e1[2] * e2[0] - e1[0] * e2[2]
    cz = e1[0] * e2[1] - e1[1] * e2[0]
    c2sum = cx * cx + cy * cy + cz * cz
    areas = 0.5 * jnp.sqrt(c2sum)
    asum = jnp.sum(areas)
    p = areas / asum + np.float32(1e-12)
    nv = np.float32(-1.0) / p
    ninvp_ref[...] = jnp.broadcast_to(nv[:, None, :],
                                      (_NCHUNK, _BS, _CHUNK))

    def cot(a, b, c):
        d1 = [b[k] - a[k] for k in range(3)]
        d2 = [c[k] - a[k] for k in range(3)]
        cosang = d1[0] * d2[0] + d1[1] * d2[1] + d1[2] * d2[2]
        sx = d1[1] * d2[2] - d1[2] * d2[1]
        sy = d1[2] * d2[0] - d1[0] * d2[2]
        sz = d1[0] * d2[1] - d1[1] * d2[0]
        sinang = jnp.sqrt(sx * sx + sy * sy + sz * sz)
        return cosang / (sinang + np.float32(1e-12))

    c0 = cot(v0, v1, v2)
    c1 = cot(v1, v2, v0)
    c2 = cot(v2, v0, v1)
    for k in range(3):
        pay_ref[k, 0] = c1 * v2[k] + c2 * v1[k]
        pay_ref[k, 1] = c0 * v2[k] + c2 * v0[k]
        pay_ref[k, 2] = c0 * v1[k] + c1 * v0[k]
    wsplat = jnp.zeros_like(c0)
    pay_ref[3, 0] = c1 + c2 + wsplat
    pay_ref[3, 1] = c0 + c2 + wsplat
    pay_ref[3, 2] = c0 + c1 + wsplat
    L = el_ref[...]

    def sq(d):
        n = jnp.sqrt(d[0] * d[0] + d[1] * d[1] + d[2] * d[2]) - L
        return n * n

    d01 = [v0[k] - v1[k] for k in range(3)]
    d12 = [v1[k] - v2[k] for k in range(3)]
    d02 = [v0[k] - v2[k] for k in range(3)]
    esum_ref[...] = jnp.sum(sq(d01) + sq(d12) + sq(d02)).reshape(1, 1)


def _prep(vcomp, edge_len):
    return pl.pallas_call(
        _prep_body,
        in_specs=[
            pl.BlockSpec(memory_space=pltpu.VMEM),
            pl.BlockSpec(memory_space=pltpu.VMEM),
        ],
        out_specs=[
            pl.BlockSpec(memory_space=pltpu.VMEM),
            pl.BlockSpec(memory_space=pltpu.VMEM),
            pl.BlockSpec(memory_space=pltpu.VMEM),
        ],
        out_shape=[
            jax.ShapeDtypeStruct((_NCHUNK, _BS, _CHUNK), jnp.float32),
            jax.ShapeDtypeStruct((4, 3, _NCHUNK, _CHUNK), jnp.float32),
            jax.ShapeDtypeStruct((1, 1), jnp.float32),
        ],
    )(vcomp, edge_len.reshape(1, 1))


_VP = 40960
_NW = 32
_P3F = 245760
_EPW = 4 * _P3F // _NW
_NCK = _EPW // 128
_ACC = 4 * _VP


def _lap_scatter_body(vals_hbm, idx_hbm, zeros_hbm, out_hbm,
                      vals_v, idx_v, acc_sh):
    c = jax.lax.axis_index("c")
    s = jax.lax.axis_index("s")
    wid = s * 2 + c
    aslc = _ACC // 16
    pltpu.sync_copy(zeros_hbm.at[pl.ds(s * aslc, aslc)],
                    acc_sh.at[pl.ds(s * aslc, aslc)])
    plsc.subcore_barrier()
    pltpu.sync_copy(vals_hbm.at[pl.ds(wid * _EPW, _EPW)], vals_v)
    pltpu.sync_copy(idx_hbm.at[pl.ds(wid * _NCK, _NCK)], idx_v)

    def chunk(j, carry):
        pltpu.sync_copy(vals_v.at[pl.ds(j * 128, 128)],
                        acc_sh.at[idx_v.at[j]], add=True)
        return carry

    jax.lax.fori_loop(0, _NCK, chunk, 0)
    plsc.subcore_barrier()
    pltpu.sync_copy(acc_sh.at[pl.ds(s * aslc, aslc)],
                    out_hbm.at[c].at[pl.ds(s * aslc, aslc)])


def _lap_scatter(vals_flat, idx2, zeros):
    kfn = pl.kernel(
        _lap_scatter_body,
        out_type=jax.ShapeDtypeStruct((2, _ACC), jnp.float32),
        mesh=plsc.VectorSubcoreMesh(core_axis_name="c", subcore_axis_name="s",
                                    num_cores=2, num_subcores=16),
        scratch_types=[
            pltpu.VMEM((_EPW,), jnp.float32),
            pltpu.VMEM((_NCK, 128), jnp.int32),
            pltpu.VMEM_SHARED((_ACC,), jnp.float32),
        ],
    )
    return kfn(vals_flat, idx2, zeros)


def _cot(a, b, c):
    e1 = b - a
    e2 = c - a
    cosang = jnp.sum(e1 * e2, axis=-1)
    sinang = jnp.linalg.norm(jnp.cross(e1, e2), axis=-1)
    return cosang / (sinang + 1e-12)


def kernel(verts_src, trg, edge_len, faces):
    f0, f1, f2 = faces[:, 0], faces[:, 1], faces[:, 2]
    v0 = verts_src[f0]
    v1 = verts_src[f1]
    v2 = verts_src[f2]
    vcomp = jnp.concatenate([v0, v1, v2], axis=1).T.reshape(9, _NCHUNK, _CHUNK)
    ninvp, pay, esum = _prep(vcomp, edge_len)
    skey = jax.random.key(42)
    _, ks2, ks3 = jax.random.split(skey, 3)
    face_idx = _sample_face_idx(ninvp)[None, :]
    u = jax.random.uniform(ks2, (B, S, 1), dtype=jnp.float32)
    vv = jax.random.uniform(ks3, (B, S, 1), dtype=jnp.float32)
    su = jnp.sqrt(u)
    w0 = 1.0 - su
    w1 = su * (1.0 - vv)
    w2 = su * vv
    sample_scr = w0 * v0[face_idx] + w1 * v1[face_idx] + w2 * v2[face_idx]
    rm, cm = _chamfer(sample_scr[0], trg[0])
    loss_p0 = jnp.mean(rm[:S]) + jnp.mean(cm[:S])
    loss_n1 = jnp.asarray(1e-5, dtype=jnp.float32)
    idx3 = jnp.concatenate(
        [f0, f1, f2,
         jnp.full((_P3F - 3 * F,), _VP - 1, jnp.int32)]).astype(jnp.int32)
    idx_el = (jnp.arange(4, dtype=jnp.int32)[:, None] * _VP
              + idx3[None, :]).reshape(-1)
    vals4 = jnp.concatenate(
        [pay.reshape(4, 3 * F),
         jnp.zeros((4, _P3F - 3 * F), jnp.float32)], axis=1)
    acc2 = _lap_scatter(vals4.reshape(-1),
                        idx_el.reshape(_NW * _NCK, 128),
                        jnp.zeros((_ACC,), jnp.float32))
    A = (acc2[0] + acc2[1]).reshape(4, _VP)
    Lv = A[:3, :V].T
    wsum = A[3, :V]
    safe = jnp.where(wsum > 0, wsum, 1.0)
    norm_w = jnp.where(wsum > 0, 1.0 / safe, 0.0)
    lap = Lv * norm_w[:, None] - verts_src
    loss_laplacian = jnp.mean(jnp.linalg.norm(lap, axis=1))
    loss_edge = esum[0, 0] / np.float32(3 * F)
    return jnp.stack([loss_p0, loss_n1, loss_laplacian, loss_edge])


# speedup vs baseline: 1.0394x; 1.0036x over previous
import functools

import jax
import jax.numpy as jnp
import numpy as np
from jax.experimental import pallas as pl
from jax.experimental.pallas import tpu as pltpu
from jax.experimental.pallas import tpu_sc as plsc

V = 40000
F = 80000
B = 1
S = 5000
SP = 5120
BI = 1024


def _i32(v):
    return np.array([v], np.uint32).view(np.int32)[0]


_K1 = np.uint32(1832780943)
_K2 = np.uint32(270669613)
_K3 = _K1 ^ _K2 ^ np.uint32(0x1BD11BDA)
_KS = (_K1, _K2, _K3)
_ROT1 = (13, 15, 26, 6)
_ROT2 = (17, 29, 16, 24)
_TINY = np.float32(np.finfo(np.float32).tiny)
_CHUNK = 640
_NCHUNK = F // _CHUNK
_BS = 8
_UNROLL = 10


def _tf_rounds(x0, x1, rots):
    for r in rots:
        x0 = x0 + x1
        x1 = jax.lax.shift_left(x1, np.int32(r)) | jax.lax.shift_right_logical(
            x1, np.int32(32 - r))
        x1 = x0 ^ x1
    return x0, x1


def _tf_bits_from_x1(x1):
    inject = ((_KS[1], _KS[2] + np.uint32(1)),
              (_KS[2], _KS[0] + np.uint32(2)),
              (_KS[0], _KS[1] + np.uint32(3)),
              (_KS[1], _KS[2] + np.uint32(4)),
              (_KS[2], _KS[0] + np.uint32(5)))
    rots = (_ROT1, _ROT2, _ROT1, _ROT2, _ROT1)
    r = _ROT1[0]
    x0 = x1 + _i32(_KS[0])
    x1r = jax.lax.shift_left(x1, np.int32(r)) | jax.lax.shift_right_logical(
        x1, np.int32(32 - r))
    x1 = x0 ^ x1r
    x0, x1 = _tf_rounds(x0, x1, _ROT1[1:])
    x0 = x0 + _i32(inject[0][0])
    x1 = x1 + _i32(inject[0][1])
    for (a, b), rr in zip(inject[1:], rots[1:]):
        x0, x1 = _tf_rounds(x0, x1, rr)
        x0 = x0 + _i32(a)
        x1 = x1 + _i32(b)
    return x0 ^ x1


def _score_chunk(ninvp_ref, base_x1, cc):
    x1 = base_x1 + cc * _CHUNK
    bits = _tf_bits_from_x1(x1)
    fb = jax.lax.shift_right_logical(bits, np.int32(9)) | _i32(0x3F800000)
    u = jax.lax.bitcast_convert_type(fb, jnp.float32) - np.float32(1.0)
    ninvp = ninvp_ref[cc]
    return jnp.log2(u) * ninvp


def _sample_body(ninvp_ref, out_ref):
    pid = pl.program_id(0)
    base_x1 = ((pid * _BS
                + jax.lax.broadcasted_iota(jnp.int32, (_BS, _CHUNK), 0)) * F
               + jax.lax.broadcasted_iota(jnp.int32, (_BS, _CHUNK), 1)
               + _i32(_KS[1]))

    def step(c, carry):
        minval, minc = carry
        for k in range(_UNROLL):
            cc = _UNROLL * c + k
            score = _score_chunk(ninvp_ref, base_x1, cc)
            upd = score < minval
            minval = jnp.where(upd, score, minval)
            minc = jnp.where(upd, cc, minc)
        return minval, minc

    init = (jnp.full((_BS, _CHUNK), np.float32(np.inf), jnp.float32),
            jnp.zeros((_BS, _CHUNK), jnp.int32))
    nloop = _NCHUNK // _UNROLL
    minval, minc = jax.lax.fori_loop(0, nloop, step, init)
    for cc in range(nloop * _UNROLL, _NCHUNK):
        score = _score_chunk(ninvp_ref, base_x1, cc)
        upd = score < minval
        minval = jnp.where(upd, score, minval)
        minc = jnp.where(upd, cc, minc)

    minidx = minc * _CHUNK + jax.lax.broadcasted_iota(
        jnp.int32, (_BS, _CHUNK), 1)
    rowmin = jnp.min(minval, axis=1, keepdims=True)
    cand = jnp.where(minval == rowmin, minidx, np.int32(2**31 - 1))
    out_ref[...] = jnp.min(cand, axis=1, keepdims=True)


def _sample_face_idx(ninvp):
    out = pl.pallas_call(
        _sample_body,
        grid=(S // _BS,),
        in_specs=[pl.BlockSpec((_NCHUNK, _BS, _CHUNK), lambda i: (0, 0, 0))],
        out_specs=pl.BlockSpec((_BS, 1), lambda i: (i, 0)),
        out_shape=jax.ShapeDtypeStruct((S, 1), jnp.int32),
        compiler_params=pltpu.CompilerParams(
            dimension_semantics=("arbitrary",)),
    )(ninvp)
    return out.reshape(S)


def _chamfer_body(s_ref, tT_ref, rm_ref, cm_ref):
    i = pl.program_id(0)
    s = s_ref[...]
    tT = tT_ref[...]
    d = jnp.zeros((BI, SP), jnp.float32)
    for c in range(3):
        diff = s[:, c:c + 1] - tT[c:c + 1, :]
        d = d + diff * diff
    rm = jnp.min(d, axis=1)
    rm_ref[...] = rm.reshape(BI // 128, 128)
    cm = jnp.min(d, axis=0)
    cm = cm.reshape(SP // 128, 128)

    @pl.when(i == 0)
    def _():
        cm_ref[...] = cm

    @pl.when(i != 0)
    def _():
        cm_ref[...] = jnp.minimum(cm_ref[...], cm)


def _chamfer(sample_pts, trg_pts):
    big_s = 1e9
    big_t = -1e9
    s = jnp.full((SP, 8), big_s, jnp.float32).at[:S, :3].set(sample_pts)
    t = jnp.full((SP, 8), big_t, jnp.float32).at[:S, :3].set(trg_pts)
    tT = t.T.reshape(8, SP)

    rm, cm = pl.pallas_call(
        _chamfer_body,
        grid=(SP // BI,),
        in_specs=[
            pl.BlockSpec((BI, 8), lambda i: (i, 0)),
            pl.BlockSpec((8, SP), lambda i: (0, 0)),
        ],
        out_specs=[
            pl.BlockSpec((BI // 128, 128), lambda i: (i, 0)),
            pl.BlockSpec((SP // 128, 128), lambda i: (0, 0)),
        ],
        out_shape=[
            jax.ShapeDtypeStruct((SP // 128, 128), jnp.float32),
            jax.ShapeDtypeStruct((SP // 128, 128), jnp.float32),
        ],
        compiler_params=pltpu.CompilerParams(
            dimension_semantics=("arbitrary",)),
    )(s, tT)
    return rm.reshape(SP), cm.reshape(SP)


def _prep_body(vc_ref, el_ref, ninvp_ref, pay_ref, esum_ref):
    vs = [vc_ref[k] for k in range(9)]
    v0 = vs[0:3]
    v1 = vs[3:6]
    v2 = vs[6:9]
    e1 = [v1[k] - v0[k] for k in range(3)]
    e2 = [v2[k] - v0[k] for k in range(3)]
    cx = e1[1] * e2[2] - e1[2] * e2[1]
    cy = e1[2] * e2[0] - e1[0] * e2[2]
    cz = e1[0] * e2[1] - e1[1] * e2[0]
    c2sum = cx * cx + cy * cy + cz * cz
    areas = 0.5 * jnp.sqrt(c2sum)
    asum = jnp.sum(areas)
    p = areas / asum + np.float32(1e-12)
    nv = np.float32(-1.0) / p
    ninvp_ref[...] = jnp.broadcast_to(nv[:, None, :],
                                      (_NCHUNK, _BS, _CHUNK))

    def cot(a, b, c):
        d1 = [b[k] - a[k] for k in range(3)]
        d2 = [c[k] - a[k] for k in range(3)]
        cosang = d1[0] * d2[0] + d1[1] * d2[1] + d1[2] * d2[2]
        sx = d1[1] * d2[2] - d1[2] * d2[1]
        sy = d1[2] * d2[0] - d1[0] * d2[2]
        sz = d1[0] * d2[1] - d1[1] * d2[0]
        sinang = jnp.sqrt(sx * sx + sy * sy + sz * sz)
        return cosang / (sinang + np.float32(1e-12))

    c0 = cot(v0, v1, v2)
    c1 = cot(v1, v2, v0)
    c2 = cot(v2, v0, v1)
    for k in range(3):
        pay_ref[k, 0] = c1 * v2[k] + c2 * v1[k]
        pay_ref[k, 1] = c0 * v2[k] + c2 * v0[k]
        pay_ref[k, 2] = c0 * v1[k] + c1 * v0[k]
    wsplat = jnp.zeros_like(c0)
    pay_ref[3, 0] = c1 + c2 + wsplat
    pay_ref[3, 1] = c0 + c2 + wsplat
    pay_ref[3, 2] = c0 + c1 + wsplat
    L = el_ref[...]

    def sq(d):
        n = jnp.sqrt(d[0] * d[0] + d[1] * d[1] + d[2] * d[2]) - L
        return n * n

    d01 = [v0[k] - v1[k] for k in range(3)]
    d12 = [v1[k] - v2[k] for k in range(3)]
    d02 = [v0[k] - v2[k] for k in range(3)]
    esum_ref[...] = jnp.sum(sq(d01) + sq(d12) + sq(d02)).reshape(1, 1)


def _prep(vcomp, edge_len):
    return pl.pallas_call(
        _prep_body,
        in_specs=[
            pl.BlockSpec(memory_space=pltpu.VMEM),
            pl.BlockSpec(memory_space=pltpu.VMEM),
        ],
        out_specs=[
            pl.BlockSpec(memory_space=pltpu.VMEM),
            pl.BlockSpec(memory_space=pltpu.VMEM),
            pl.BlockSpec(memory_space=pltpu.VMEM),
        ],
        out_shape=[
            jax.ShapeDtypeStruct((_NCHUNK, _BS, _CHUNK), jnp.float32),
            jax.ShapeDtypeStruct((4, 3, _NCHUNK, _CHUNK), jnp.float32),
            jax.ShapeDtypeStruct((1, 1), jnp.float32),
        ],
    )(vcomp, edge_len.reshape(1, 1))


_VP = 40960
_NW = 32
_P3F = 245760
_EPW = 4 * _P3F // _NW
_NCK = _EPW // 128
_ACC = 4 * _VP


def _lap_scatter_body(vals_hbm, idx_hbm, zeros_hbm, out_hbm,
                      vals_v, idx_v, acc_sh):
    c = jax.lax.axis_index("c")
    s = jax.lax.axis_index("s")
    wid = s * 2 + c
    aslc = _ACC // 16
    pltpu.sync_copy(zeros_hbm.at[pl.ds(s * aslc, aslc)],
                    acc_sh.at[pl.ds(s * aslc, aslc)])
    plsc.subcore_barrier()
    pltpu.sync_copy(vals_hbm.at[pl.ds(wid * _EPW, _EPW)], vals_v)
    pltpu.sync_copy(idx_hbm.at[pl.ds(wid * _NCK, _NCK)], idx_v)

    def chunk(j, carry):
        pltpu.sync_copy(vals_v.at[pl.ds(j * 128, 128)],
                        acc_sh.at[idx_v.at[j]], add=True)
        return carry

    jax.lax.fori_loop(0, _NCK, chunk, 0)
    plsc.subcore_barrier()
    pltpu.sync_copy(acc_sh.at[pl.ds(s * aslc, aslc)],
                    out_hbm.at[c].at[pl.ds(s * aslc, aslc)])


def _lap_scatter(vals_flat, idx2, zeros):
    kfn = pl.kernel(
        _lap_scatter_body,
        out_type=jax.ShapeDtypeStruct((2, _ACC), jnp.float32),
        mesh=plsc.VectorSubcoreMesh(core_axis_name="c", subcore_axis_name="s",
                                    num_cores=2, num_subcores=16),
        scratch_types=[
            pltpu.VMEM((_EPW,), jnp.float32),
            pltpu.VMEM((_NCK, 128), jnp.int32),
            pltpu.VMEM_SHARED((_ACC,), jnp.float32),
        ],
    )
    return kfn(vals_flat, idx2, zeros)


def _cot(a, b, c):
    e1 = b - a
    e2 = c - a
    cosang = jnp.sum(e1 * e2, axis=-1)
    sinang = jnp.linalg.norm(jnp.cross(e1, e2), axis=-1)
    return cosang / (sinang + 1e-12)


def kernel(verts_src, trg, edge_len, faces):
    f0, f1, f2 = faces[:, 0], faces[:, 1], faces[:, 2]
    v0 = verts_src[f0]
    v1 = verts_src[f1]
    v2 = verts_src[f2]
    vcomp = jnp.concatenate([v0, v1, v2], axis=1).T.reshape(9, _NCHUNK, _CHUNK)
    ninvp, pay, esum = _prep(vcomp, edge_len)
    skey = jax.random.key(42)
    _, ks2, ks3 = jax.random.split(skey, 3)
    face_idx = _sample_face_idx(ninvp)[None, :]
    u = jax.random.uniform(ks2, (B, S, 1), dtype=jnp.float32)
    vv = jax.random.uniform(ks3, (B, S, 1), dtype=jnp.float32)
    su = jnp.sqrt(u)
    w0 = 1.0 - su
    w1 = su * (1.0 - vv)
    w2 = su * vv
    sample_scr = w0 * v0[face_idx] + w1 * v1[face_idx] + w2 * v2[face_idx]
    rm, cm = _chamfer(sample_scr[0], trg[0])
    loss_p0 = jnp.mean(rm[:S]) + jnp.mean(cm[:S])
    loss_n1 = jnp.asarray(1e-5, dtype=jnp.float32)
    idx3 = jnp.concatenate(
        [f0, f1, f2,
         jnp.full((_P3F - 3 * F,), _VP - 1, jnp.int32)]).astype(jnp.int32)
    idx_el = (jnp.arange(4, dtype=jnp.int32)[:, None] * _VP
              + idx3[None, :]).reshape(-1)
    vals4 = jnp.concatenate(
        [pay.reshape(4, 3 * F),
         jnp.zeros((4, _P3F - 3 * F), jnp.float32)], axis=1)
    acc2 = _lap_scatter(vals4.reshape(-1),
                        idx_el.reshape(_NW * _NCK, 128),
                        jnp.zeros((_ACC,), jnp.float32))
    A = (acc2[0] + acc2[1]).reshape(4, _VP)
    Lv = A[:3, :V].T
    wsum = A[3, :V]
    safe = jnp.where(wsum > 0, wsum, 1.0)
    norm_w = jnp.where(wsum > 0, 1.0 / safe, 0.0)
    lap = Lv * norm_w[:, None] - verts_src
    loss_laplacian = jnp.mean(jnp.linalg.norm(lap, axis=1))
    loss_edge = esum[0, 0] / np.float32(3 * F)
    return jnp.stack([loss_p0, loss_n1, loss_laplacian, loss_edge])


# speedup vs baseline: 1.0403x; 1.0009x over previous
import functools

import jax
import jax.numpy as jnp
import numpy as np
from jax.experimental import pallas as pl
from jax.experimental.pallas import tpu as pltpu
from jax.experimental.pallas import tpu_sc as plsc

V = 40000
F = 80000
B = 1
S = 5000
SP = 5120
BI = 1024


def _i32(v):
    return np.array([v], np.uint32).view(np.int32)[0]


_K1 = np.uint32(1832780943)
_K2 = np.uint32(270669613)
_K3 = _K1 ^ _K2 ^ np.uint32(0x1BD11BDA)
_KS = (_K1, _K2, _K3)
_ROT1 = (13, 15, 26, 6)
_ROT2 = (17, 29, 16, 24)
_TINY = np.float32(np.finfo(np.float32).tiny)
_CHUNK = 640
_NCHUNK = F // _CHUNK
_BS = 8
_UNROLL = 12


def _tf_rounds(x0, x1, rots):
    for r in rots:
        x0 = x0 + x1
        x1 = jax.lax.shift_left(x1, np.int32(r)) | jax.lax.shift_right_logical(
            x1, np.int32(32 - r))
        x1 = x0 ^ x1
    return x0, x1


def _tf_bits_from_x1(x1):
    inject = ((_KS[1], _KS[2] + np.uint32(1)),
              (_KS[2], _KS[0] + np.uint32(2)),
              (_KS[0], _KS[1] + np.uint32(3)),
              (_KS[1], _KS[2] + np.uint32(4)),
              (_KS[2], _KS[0] + np.uint32(5)))
    rots = (_ROT1, _ROT2, _ROT1, _ROT2, _ROT1)
    r = _ROT1[0]
    x0 = x1 + _i32(_KS[0])
    x1r = jax.lax.shift_left(x1, np.int32(r)) | jax.lax.shift_right_logical(
        x1, np.int32(32 - r))
    x1 = x0 ^ x1r
    x0, x1 = _tf_rounds(x0, x1, _ROT1[1:])
    x0 = x0 + _i32(inject[0][0])
    x1 = x1 + _i32(inject[0][1])
    for (a, b), rr in zip(inject[1:], rots[1:]):
        x0, x1 = _tf_rounds(x0, x1, rr)
        x0 = x0 + _i32(a)
        x1 = x1 + _i32(b)
    return x0 ^ x1


def _score_chunk(ninvp_ref, base_x1, cc):
    x1 = base_x1 + cc * _CHUNK
    bits = _tf_bits_from_x1(x1)
    fb = jax.lax.shift_right_logical(bits, np.int32(9)) | _i32(0x3F800000)
    u = jax.lax.bitcast_convert_type(fb, jnp.float32) - np.float32(1.0)
    ninvp = ninvp_ref[cc]
    return jnp.log2(u) * ninvp


def _sample_body(ninvp_ref, out_ref):
    pid = pl.program_id(0)
    base_x1 = ((pid * _BS
                + jax.lax.broadcasted_iota(jnp.int32, (_BS, _CHUNK), 0)) * F
               + jax.lax.broadcasted_iota(jnp.int32, (_BS, _CHUNK), 1)
               + _i32(_KS[1]))

    def step(c, carry):
        minval, minc = carry
        for k in range(_UNROLL):
            cc = _UNROLL * c + k
            score = _score_chunk(ninvp_ref, base_x1, cc)
            upd = score < minval
            minval = jnp.where(upd, score, minval)
            minc = jnp.where(upd, cc, minc)
        return minval, minc

    init = (jnp.full((_BS, _CHUNK), np.float32(np.inf), jnp.float32),
            jnp.zeros((_BS, _CHUNK), jnp.int32))
    nloop = _NCHUNK // _UNROLL
    minval, minc = jax.lax.fori_loop(0, nloop, step, init)
    for cc in range(nloop * _UNROLL, _NCHUNK):
        score = _score_chunk(ninvp_ref, base_x1, cc)
        upd = score < minval
        minval = jnp.where(upd, score, minval)
        minc = jnp.where(upd, cc, minc)

    minidx = minc * _CHUNK + jax.lax.broadcasted_iota(
        jnp.int32, (_BS, _CHUNK), 1)
    rowmin = jnp.min(minval, axis=1, keepdims=True)
    cand = jnp.where(minval == rowmin, minidx, np.int32(2**31 - 1))
    out_ref[...] = jnp.min(cand, axis=1, keepdims=True)


def _sample_face_idx(ninvp):
    out = pl.pallas_call(
        _sample_body,
        grid=(S // _BS,),
        in_specs=[pl.BlockSpec((_NCHUNK, _BS, _CHUNK), lambda i: (0, 0, 0))],
        out_specs=pl.BlockSpec((_BS, 1), lambda i: (i, 0)),
        out_shape=jax.ShapeDtypeStruct((S, 1), jnp.int32),
        compiler_params=pltpu.CompilerParams(
            dimension_semantics=("arbitrary",)),
    )(ninvp)
    return out.reshape(S)


def _chamfer_body(s_ref, tT_ref, rm_ref, cm_ref):
    i = pl.program_id(0)
    s = s_ref[...]
    tT = tT_ref[...]
    d = jnp.zeros((BI, SP), jnp.float32)
    for c in range(3):
        diff = s[:, c:c + 1] - tT[c:c + 1, :]
        d = d + diff * diff
    rm = jnp.min(d, axis=1)
    rm_ref[...] = rm.reshape(BI // 128, 128)
    cm = jnp.min(d, axis=0)
    cm = cm.reshape(SP // 128, 128)

    @pl.when(i == 0)
    def _():
        cm_ref[...] = cm

    @pl.when(i != 0)
    def _():
        cm_ref[...] = jnp.minimum(cm_ref[...], cm)


def _chamfer(sample_pts, trg_pts):
    big_s = 1e9
    big_t = -1e9
    s = jnp.full((SP, 8), big_s, jnp.float32).at[:S, :3].set(sample_pts)
    t = jnp.full((SP, 8), big_t, jnp.float32).at[:S, :3].set(trg_pts)
    tT = t.T.reshape(8, SP)

    rm, cm = pl.pallas_call(
        _chamfer_body,
        grid=(SP // BI,),
        in_specs=[
            pl.BlockSpec((BI, 8), lambda i: (i, 0)),
            pl.BlockSpec((8, SP), lambda i: (0, 0)),
        ],
        out_specs=[
            pl.BlockSpec((BI // 128, 128), lambda i: (i, 0)),
            pl.BlockSpec((SP // 128, 128), lambda i: (0, 0)),
        ],
        out_shape=[
            jax.ShapeDtypeStruct((SP // 128, 128), jnp.float32),
            jax.ShapeDtypeStruct((SP // 128, 128), jnp.float32),
        ],
        compiler_params=pltpu.CompilerParams(
            dimension_semantics=("arbitrary",)),
    )(s, tT)
    return rm.reshape(SP), cm.reshape(SP)


def _prep_body(vc_ref, el_ref, ninvp_ref, pay_ref, esum_ref):
    vs = [vc_ref[k] for k in range(9)]
    v0 = vs[0:3]
    v1 = vs[3:6]
    v2 = vs[6:9]
    e1 = [v1[k] - v0[k] for k in range(3)]
    e2 = [v2[k] - v0[k] for k in range(3)]
    cx = e1[1] * e2[2] - e1[2] * e2[1]
    cy = e1[2] * e2[0] - e1[0] * e2[2]
    cz = e1[0] * e2[1] - e1[1] * e2[0]
    c2sum = cx * cx + cy * cy + cz * cz
    areas = 0.5 * jnp.sqrt(c2sum)
    asum = jnp.sum(areas)
    p = areas / asum + np.float32(1e-12)
    nv = np.float32(-1.0) / p
    ninvp_ref[...] = jnp.broadcast_to(nv[:, None, :],
                                      (_NCHUNK, _BS, _CHUNK))

    def cot(a, b, c):
        d1 = [b[k] - a[k] for k in range(3)]
        d2 = [c[k] - a[k] for k in range(3)]
        cosang = d1[0] * d2[0] + d1[1] * d2[1] + d1[2] * d2[2]
        sx = d1[1] * d2[2] - d1[2] * d2[1]
        sy = d1[2] * d2[0] - d1[0] * d2[2]
        sz = d1[0] * d2[1] - d1[1] * d2[0]
        sinang = jnp.sqrt(sx * sx + sy * sy + sz * sz)
        return cosang / (sinang + np.float32(1e-12))

    c0 = cot(v0, v1, v2)
    c1 = cot(v1, v2, v0)
    c2 = cot(v2, v0, v1)
    for k in range(3):
        pay_ref[k, 0] = c1 * v2[k] + c2 * v1[k]
        pay_ref[k, 1] = c0 * v2[k] + c2 * v0[k]
        pay_ref[k, 2] = c0 * v1[k] + c1 * v0[k]
    wsplat = jnp.zeros_like(c0)
    pay_ref[3, 0] = c1 + c2 + wsplat
    pay_ref[3, 1] = c0 + c2 + wsplat
    pay_ref[3, 2] = c0 + c1 + wsplat
    L = el_ref[...]

    def sq(d):
        n = jnp.sqrt(d[0] * d[0] + d[1] * d[1] + d[2] * d[2]) - L
        return n * n

    d01 = [v0[k] - v1[k] for k in range(3)]
    d12 = [v1[k] - v2[k] for k in range(3)]
    d02 = [v0[k] - v2[k] for k in range(3)]
    esum_ref[...] = jnp.sum(sq(d01) + sq(d12) + sq(d02)).reshape(1, 1)


def _prep(vcomp, edge_len):
    return pl.pallas_call(
        _prep_body,
        in_specs=[
            pl.BlockSpec(memory_space=pltpu.VMEM),
            pl.BlockSpec(memory_space=pltpu.VMEM),
        ],
        out_specs=[
            pl.BlockSpec(memory_space=pltpu.VMEM),
            pl.BlockSpec(memory_space=pltpu.VMEM),
            pl.BlockSpec(memory_space=pltpu.VMEM),
        ],
        out_shape=[
            jax.ShapeDtypeStruct((_NCHUNK, _BS, _CHUNK), jnp.float32),
            jax.ShapeDtypeStruct((4, 3, _NCHUNK, _CHUNK), jnp.float32),
            jax.ShapeDtypeStruct((1, 1), jnp.float32),
        ],
    )(vcomp, edge_len.reshape(1, 1))


_VP = 40960
_NW = 32
_P3F = 245760
_EPW = 4 * _P3F // _NW
_NCK = _EPW // 128
_ACC = 4 * _VP


def _lap_scatter_body(vals_hbm, idx_hbm, zeros_hbm, out_hbm,
                      vals_v, idx_v, acc_sh):
    c = jax.lax.axis_index("c")
    s = jax.lax.axis_index("s")
    wid = s * 2 + c
    aslc = _ACC // 16
    pltpu.sync_copy(zeros_hbm.at[pl.ds(s * aslc, aslc)],
                    acc_sh.at[pl.ds(s * aslc, aslc)])
    plsc.subcore_barrier()
    pltpu.sync_copy(vals_hbm.at[pl.ds(wid * _EPW, _EPW)], vals_v)
    pltpu.sync_copy(idx_hbm.at[pl.ds(wid * _NCK, _NCK)], idx_v)

    def chunk(j, carry):
        pltpu.sync_copy(vals_v.at[pl.ds(j * 128, 128)],
                        acc_sh.at[idx_v.at[j]], add=True)
        return carry

    jax.lax.fori_loop(0, _NCK, chunk, 0)
    plsc.subcore_barrier()
    pltpu.sync_copy(acc_sh.at[pl.ds(s * aslc, aslc)],
                    out_hbm.at[c].at[pl.ds(s * aslc, aslc)])


def _lap_scatter(vals_flat, idx2, zeros):
    kfn = pl.kernel(
        _lap_scatter_body,
        out_type=jax.ShapeDtypeStruct((2, _ACC), jnp.float32),
        mesh=plsc.VectorSubcoreMesh(core_axis_name="c", subcore_axis_name="s",
                                    num_cores=2, num_subcores=16),
        scratch_types=[
            pltpu.VMEM((_EPW,), jnp.float32),
            pltpu.VMEM((_NCK, 128), jnp.int32),
            pltpu.VMEM_SHARED((_ACC,), jnp.float32),
        ],
    )
    return kfn(vals_flat, idx2, zeros)


def _cot(a, b, c):
    e1 = b - a
    e2 = c - a
    cosang = jnp.sum(e1 * e2, axis=-1)
    sinang = jnp.linalg.norm(jnp.cross(e1, e2), axis=-1)
    return cosang / (sinang + 1e-12)


def kernel(verts_src, trg, edge_len, faces):
    f0, f1, f2 = faces[:, 0], faces[:, 1], faces[:, 2]
    v0 = verts_src[f0]
    v1 = verts_src[f1]
    v2 = verts_src[f2]
    vcomp = jnp.concatenate([v0, v1, v2], axis=1).T.reshape(9, _NCHUNK, _CHUNK)
    ninvp, pay, esum = _prep(vcomp, edge_len)
    skey = jax.random.key(42)
    _, ks2, ks3 = jax.random.split(skey, 3)
    face_idx = _sample_face_idx(ninvp)[None, :]
    u = jax.random.uniform(ks2, (B, S, 1), dtype=jnp.float32)
    vv = jax.random.uniform(ks3, (B, S, 1), dtype=jnp.float32)
    su = jnp.sqrt(u)
    w0 = 1.0 - su
    w1 = su * (1.0 - vv)
    w2 = su * vv
    sample_scr = w0 * v0[face_idx] + w1 * v1[face_idx] + w2 * v2[face_idx]
    rm, cm = _chamfer(sample_scr[0], trg[0])
    loss_p0 = jnp.mean(rm[:S]) + jnp.mean(cm[:S])
    loss_n1 = jnp.asarray(1e-5, dtype=jnp.float32)
    idx3 = jnp.concatenate(
        [f0, f1, f2,
         jnp.full((_P3F - 3 * F,), _VP - 1, jnp.int32)]).astype(jnp.int32)
    idx_el = (jnp.arange(4, dtype=jnp.int32)[:, None] * _VP
              + idx3[None, :]).reshape(-1)
    vals4 = jnp.concatenate(
        [pay.reshape(4, 3 * F),
         jnp.zeros((4, _P3F - 3 * F), jnp.float32)], axis=1)
    acc2 = _lap_scatter(vals4.reshape(-1),
                        idx_el.reshape(_NW * _NCK, 128),
                        jnp.zeros((_ACC,), jnp.float32))
    A = (acc2[0] + acc2[1]).reshape(4, _VP)
    Lv = A[:3, :V].T
    wsum = A[3, :V]
    safe = jnp.where(wsum > 0, wsum, 1.0)
    norm_w = jnp.where(wsum > 0, 1.0 / safe, 0.0)
    lap = Lv * norm_w[:, None] - verts_src
    loss_laplacian = jnp.mean(jnp.linalg.norm(lap, axis=1))
    loss_edge = esum[0, 0] / np.float32(3 * F)
    return jnp.stack([loss_p0, loss_n1, loss_laplacian, loss_edge])
